# Initial kernel scaffold; baseline (speedup 1.0000x reference)
import jax
import jax.numpy as jnp
from jax.experimental import pallas as pl


def kernel(node_features, edge_index, edge_sh, edge_radial, W1, b1, W2, b2, W3, b3, L0, L1):
    raise NotImplementedError("write your pallas kernel here")


# speedup vs baseline: 3.6713x; 3.6713x over previous
import functools

import numpy as np
import jax
import jax.numpy as jnp
from jax import lax
from jax.experimental import pallas as pl
from jax.experimental.pallas import tpu as pltpu
from jax.experimental.pallas import tpu_sc as plsc

MUL0, MUL1 = 16, 8
DF = MUL0 + 3 * MUL1
ALPHA = 1.0 / np.sqrt(MUL0 + MUL1)

NC, NS = 2, 16
NW = NC * NS
CH = 128
GRP = 16
ROWS = GRP * CH


def _build_maps():
    a_ev = np.zeros((3, 24), np.float32)
    for i in range(3):
        a_ev[i, i::3] = 1.0
    r_vv = np.zeros((24, 8), np.float32)
    for j in range(24):
        r_vv[j, j // 3] = 1.0
    a_q0 = np.zeros((MUL0, 256), np.float32)
    for u in range(MUL0):
        a_q0[u, u * 16:(u + 1) * 16] = 1.0
    a_vvd = np.zeros((MUL1, 128), np.float32)
    for u in range(MUL1):
        a_vvd[u, u * 16:(u + 1) * 16] = 1.0
    a_s8 = np.zeros((MUL0, 128), np.float32)
    for u in range(MUL0):
        a_s8[u, u * 8:(u + 1) * 8] = 1.0
    a_xv = np.zeros((24, 192), np.float32)
    for u in range(MUL1):
        for i in range(3):
            a_xv[3 * u + i, i * 64 + u * 8:i * 64 + (u + 1) * 8] = 1.0
    r_m0 = np.zeros((384, MUL0), np.float32)
    for u in range(MUL0):
        for w in range(MUL0):
            r_m0[u * 16 + w, w] = ALPHA
    for u in range(MUL1):
        for w in range(MUL0):
            r_m0[256 + u * 16 + w, w] = ALPHA / np.sqrt(3.0)
    r_c1 = np.zeros((128, 24), np.float32)
    for u in range(MUL0):
        for w in range(MUL1):
            for i in range(3):
                r_c1[u * 8 + w, 3 * w + i] = ALPHA
    r_d = np.zeros((192, 24), np.float32)
    for i in range(3):
        for u in range(MUL1):
            for w in range(MUL1):
                r_d[i * 64 + u * 8 + w, 3 * w + i] = ALPHA
    return a_ev, r_vv, a_q0, a_vvd, a_s8, a_xv, r_m0, r_c1, r_d


_MAPS = _build_maps()


def _msg_body(r_ref, x_ref, sh_ref, w1_ref, b1_ref, w2_ref, b2_ref, w3_ref,
              b3_ref, aev_ref, rvv_ref, aq0_ref, avvd_ref, as8_ref, axv_ref,
              rm0_ref, rc1_ref, rd_ref, out_ref):
    f32 = jnp.float32
    h = jax.nn.silu(jnp.dot(r_ref[...], w1_ref[...],
                            preferred_element_type=f32) + b1_ref[...])
    h = jax.nn.silu(jnp.dot(h, w2_ref[...],
                            preferred_element_type=f32) + b2_ref[...])
    w = jnp.dot(h, w3_ref[...], preferred_element_type=f32) + b3_ref[...]
    x = x_ref[...]
    sh = sh_ref[...]
    s = x[:, :MUL0]
    xv = x[:, MUL0:DF]
    es = sh[:, 0:1]
    ev = sh[:, 1:4]
    ev24 = jnp.dot(ev, aev_ref[...], preferred_element_type=f32)
    vvd = jnp.dot(xv * ev24, rvv_ref[...], preferred_element_type=f32)
    q16 = jnp.dot(s * es, aq0_ref[...], preferred_element_type=f32)
    v16 = jnp.dot(vvd, avvd_ref[...], preferred_element_type=f32)
    pm0 = jnp.concatenate([w[:, :256] * q16, w[:, 256:384] * v16], axis=1)
    m0 = jnp.dot(pm0, rm0_ref[...], preferred_element_type=f32)
    s8 = jnp.dot(s, as8_ref[...], preferred_element_type=f32)
    c1 = jnp.dot(w[:, 384:512] * s8, rc1_ref[...], preferred_element_type=f32)
    xv3 = jnp.dot(xv, axv_ref[...], preferred_element_type=f32)
    wv = w[:, 512:576]
    wv3 = jnp.concatenate([wv, wv, wv], axis=1)
    d = jnp.dot(wv3 * xv3, rd_ref[...], preferred_element_type=f32)
    m1 = c1 * ev24 + es * d
    out_ref[...] = jnp.concatenate([m0, m1], axis=1)


def _messages_tc(rp, x_pad, shp, W1, b1, W2, b2, W3, b3, block_e):
    e_pad = rp.shape[0]
    grid = (e_pad // block_e,)
    edge_spec = lambda cols: pl.BlockSpec((block_e, cols), lambda i: (i, 0))
    full_spec = lambda shp_: pl.BlockSpec(shp_, lambda i: tuple(0 for _ in shp_))
    maps = tuple(jnp.asarray(m) for m in _MAPS)
    in_specs = [edge_spec(8), edge_spec(DF), edge_spec(4)]
    operands = [rp, x_pad, shp,
                W1, b1.reshape(1, -1), W2, b2.reshape(1, -1),
                W3, b3.reshape(1, -1), *maps]
    for op in operands[3:]:
        in_specs.append(full_spec(op.shape))
    return pl.pallas_call(
        _msg_body,
        grid=grid,
        in_specs=in_specs,
        out_specs=pl.BlockSpec((block_e, DF), lambda i: (i, 0)),
        out_shape=jax.ShapeDtypeStruct((e_pad, DF), jnp.float32),
        compiler_params=pltpu.CompilerParams(
            dimension_semantics=("arbitrary",)),
    )(*operands)


def _make_gather_sc(n_nodes, e_pad):
    per_w = e_pad // NW
    chunks_w = per_w // CH
    groups = chunks_w // GRP
    mesh = plsc.VectorSubcoreMesh(core_axis_name="c", subcore_axis_name="s")

    @functools.partial(
        pl.kernel,
        mesh=mesh,
        out_type=jax.ShapeDtypeStruct((e_pad, DF), jnp.float32),
        scratch_types=[
            pltpu.VMEM((chunks_w, CH), jnp.int32),
            pltpu.VMEM((ROWS, DF), jnp.float32),
            pltpu.SemaphoreType.DMA,
        ],
        compiler_params=pltpu.CompilerParams(use_tc_tiling_on_sc=False),
    )
    def gather_k(nf_hbm, src2_hbm, x_hbm, idx_v, rows_v, sem):
        cid = lax.axis_index("c")
        sid = lax.axis_index("s")
        wid = sid * NC + cid
        pltpu.sync_copy(src2_hbm.at[pl.ds(wid * chunks_w, chunks_w)], idx_v)

        def grp_body(g, carry):
            descs = []
            for j in range(GRP):
                descs.append(pltpu.async_copy(
                    nf_hbm.at[idx_v.at[g * GRP + j]],
                    rows_v.at[pl.ds(j * CH, CH)], sem))
            for dsc in descs:
                dsc.wait()
            pltpu.sync_copy(rows_v,
                            x_hbm.at[pl.ds(wid * per_w + g * ROWS, ROWS)])
            return carry

        lax.fori_loop(0, groups, grp_body, 0)

    return gather_k


def _make_scatter_sc(n_nodes, e_pad):
    per_w = e_pad // NW
    chunks_w = per_w // CH
    groups = chunks_w // GRP
    rows_per_tile = n_nodes // NS
    mesh = plsc.VectorSubcoreMesh(core_axis_name="c", subcore_axis_name="s")

    @functools.partial(
        pl.kernel,
        mesh=mesh,
        out_type=jax.ShapeDtypeStruct((NC, n_nodes, DF), jnp.float32),
        scratch_types=[
            pltpu.VMEM((chunks_w, CH), jnp.int32),
            pltpu.VMEM((ROWS, DF), jnp.float32),
            pltpu.VMEM_SHARED((n_nodes, DF), jnp.float32),
            pltpu.SemaphoreType.DMA,
        ],
        compiler_params=pltpu.CompilerParams(use_tc_tiling_on_sc=False),
    )
    def scatter_k(msg_hbm, dst2_hbm, zero_hbm, agg_hbm, idx_v, msg_v, agg_sh,
                  sem):
        cid = lax.axis_index("c")
        sid = lax.axis_index("s")
        wid = sid * NC + cid

        @pl.when(sid == 0)
        def _():
            pltpu.sync_copy(zero_hbm, agg_sh)

        plsc.subcore_barrier()
        pltpu.sync_copy(dst2_hbm.at[pl.ds(wid * chunks_w, chunks_w)], idx_v)

        def grp_body(g, carry):
            pltpu.sync_copy(msg_hbm.at[pl.ds(wid * per_w + g * ROWS, ROWS)],
                            msg_v)
            for j in range(GRP):
                pltpu.sync_copy(msg_v.at[pl.ds(j * CH, CH)],
                                agg_sh.at[idx_v.at[g * GRP + j]], add=True)
            return carry

        lax.fori_loop(0, groups, grp_body, 0)
        plsc.subcore_barrier()
        pltpu.sync_copy(agg_sh.at[pl.ds(sid * rows_per_tile, rows_per_tile)],
                        agg_hbm.at[cid, pl.ds(sid * rows_per_tile,
                                              rows_per_tile)])

    return scatter_k


def _out_body(agg_ref, nf_ref, ws_ref, o_ref):
    a = agg_ref[0] + agg_ref[1]
    o_ref[...] = (jnp.dot(a, ws_ref[...], preferred_element_type=jnp.float32)
                  + nf_ref[...])


def _combine_tc(agg2, node_features, w_self):
    n = node_features.shape[0]
    return pl.pallas_call(
        _out_body,
        out_shape=jax.ShapeDtypeStruct((n, DF), jnp.float32),
    )(agg2, node_features, w_self)


def kernel(node_features, edge_index, edge_sh, edge_radial, W1, b1, W2, b2,
           W3, b3, L0, L1):
    n_nodes = node_features.shape[0]
    n_edges = edge_index.shape[1]
    quantum = NW * GRP * CH
    e_pad = ((n_edges + quantum - 1) // quantum) * quantum
    pad = e_pad - n_edges

    src2 = jnp.pad(edge_index[0], (0, pad)).reshape(-1, CH)
    dst2 = jnp.pad(edge_index[1], (0, pad)).reshape(-1, CH)
    shp = jnp.pad(edge_sh, ((0, pad), (0, 0)))
    rp = jnp.pad(edge_radial, ((0, pad), (0, 0)))

    x_pad = _make_gather_sc(n_nodes, e_pad)(node_features, src2)
    msgs = _messages_tc(rp, x_pad, shp, W1, b1, W2, b2, W3, b3, block_e=2048)
    zeros = jnp.zeros((n_nodes, DF), jnp.float32)
    agg2 = _make_scatter_sc(n_nodes, e_pad)(msgs, dst2, zeros)

    w_self = jnp.zeros((DF, DF), jnp.float32)
    w_self = w_self.at[:MUL0, :MUL0].set(L0 / np.sqrt(MUL0))
    w_self = w_self.at[MUL0:, MUL0:].set(
        jnp.kron(L1, jnp.eye(3, dtype=jnp.float32)) / np.sqrt(MUL1))
    return _combine_tc(agg2, node_features, w_self)
